# baseline (speedup 1.0000x reference)
import jax
import jax.numpy as jnp
from jax import lax
from jax.experimental import pallas as pl
from jax.experimental.pallas import tpu as pltpu
from jax.experimental.pallas import tpu_sc as plsc

TC_BLOCK_ROWS = 2048


def _tc_copy_block(x_ref, o_ref):
    o_ref[...] = x_ref[...]


def _sc_sizes_body(n, sizes_hbm, buf, sem):
    wid = lax.axis_index("s") * 2 + lax.axis_index("c")

    @pl.when(wid == 0)
    def _():
        buf[...] = jnp.full((16,), n, dtype=jnp.int32)
        pltpu.make_async_copy(buf.at[pl.ds(0, 1)], sizes_hbm, sem).start()
        pltpu.make_async_copy(buf.at[pl.ds(0, 1)], sizes_hbm, sem).wait()


def kernel(x):
    n, d = x.shape

    mesh = plsc.VectorSubcoreMesh(core_axis_name="c", subcore_axis_name="s")
    import functools
    sc_sizes = pl.kernel(
        functools.partial(_sc_sizes_body, n),
        mesh=mesh,
        out_type=jax.ShapeDtypeStruct((1,), jnp.int32),
        scratch_types=[
            pltpu.VMEM((16,), jnp.int32),
            pltpu.SemaphoreType.DMA,
        ],
    )
    sizes = sc_sizes()

    gathered = pl.pallas_call(
        _tc_copy_block,
        grid=(n // TC_BLOCK_ROWS,),
        in_specs=[pl.BlockSpec((TC_BLOCK_ROWS, d), lambda i: (i, 0))],
        out_specs=pl.BlockSpec((TC_BLOCK_ROWS, d), lambda i: (i, 0)),
        out_shape=jax.ShapeDtypeStruct((n, d), x.dtype),
    )(x)

    return (gathered, sizes)


# speedup vs baseline: 1.1891x; 1.1891x over previous
import jax
import jax.numpy as jnp
from jax.experimental import pallas as pl
from jax.experimental.pallas import tpu as pltpu

BLOCK_ROWS = 2048


def _copy_body(x_ref, o_ref, sizes_ref):
    @pl.when(pl.program_id(0) == 0)
    def _():
        sizes_ref[0] = jnp.int32(pl.num_programs(0) * BLOCK_ROWS)

    o_ref[...] = x_ref[...]


def kernel(x):
    n, d = x.shape
    gathered, sizes = pl.pallas_call(
        _copy_body,
        grid=(n // BLOCK_ROWS,),
        in_specs=[pl.BlockSpec((BLOCK_ROWS, d), lambda i: (i, 0))],
        out_specs=[
            pl.BlockSpec((BLOCK_ROWS, d), lambda i: (i, 0)),
            pl.BlockSpec(memory_space=pltpu.MemorySpace.SMEM),
        ],
        out_shape=[
            jax.ShapeDtypeStruct((n, d), x.dtype),
            jax.ShapeDtypeStruct((1,), jnp.int32),
        ],
    )(x)
    return (gathered, sizes)
